# Initial kernel scaffold; baseline (speedup 1.0000x reference)
import jax
import jax.numpy as jnp
from jax.experimental import pallas as pl


def kernel(x, rbf, idx_i, idx_j, W_rbf, W_edge, b_edge):
    raise NotImplementedError("write your pallas kernel here")


# speedup vs baseline: 1.0162x; 1.0162x over previous
import functools

import jax
import jax.numpy as jnp
from jax import lax
from jax.experimental import pallas as pl
from jax.experimental.pallas import tpu as pltpu
from jax.experimental.pallas import tpu_sc as plsc

N_NODES = 10000
N_EDGES = 320000
D = 128
NR = 16

NC = 2
NS = 16
NW = NC * NS
EDGES_PER_WORKER = N_EDGES // NW
CHUNK = 80
N_CHUNKS = EDGES_PER_WORKER // CHUNK
EB = 4
LANES = 16


def _precompute_body(x_ref, wrbf_ref, wedge_ref, b_ref, tj_ref, ti_ref, wr_ref):
    x = x_ref[...]
    tj_ref[...] = jnp.dot(x, wedge_ref[0:D, :], preferred_element_type=jnp.float32)
    ti_ref[...] = (
        jnp.dot(x, wedge_ref[D:2 * D, :], preferred_element_type=jnp.float32)
        + b_ref[...]
    )
    wr_ref[...] = jnp.dot(wrbf_ref[...], wedge_ref[2 * D:3 * D, :],
                          preferred_element_type=jnp.float32)


def _precompute(x, W_rbf, W_edge, b_edge):
    return pl.pallas_call(
        _precompute_body,
        out_shape=[
            jax.ShapeDtypeStruct((N_NODES, D), jnp.float32),
            jax.ShapeDtypeStruct((N_NODES, D), jnp.float32),
            jax.ShapeDtypeStruct((NR, D), jnp.float32),
        ],
    )(x, W_rbf, W_edge, b_edge.reshape(1, D))


def _lane_broadcast(vec, idxs):
    dnums = lax.GatherDimensionNumbers(
        offset_dims=(), collapsed_slice_dims=(0,), start_index_map=(0,))
    return lax.gather(vec, idxs[:, None], dnums, slice_sizes=(1,),
                      mode=lax.GatherScatterMode.PROMISE_IN_BOUNDS)


def _edge_body(tj_hbm, ti_hbm, wr_hbm, rbf_hbm, idxj_hbm, idxi_hbm, out_hbm,
               wr_v, idxj_v, idxi_v, rbf_v, rows_j, rows_i, out_v, sem):
    wid = lax.axis_index("s") * NC + lax.axis_index("c")
    base_w = wid * EDGES_PER_WORKER

    pltpu.sync_copy(wr_hbm, wr_v)

    def chunk_body(ci, _):
        base = base_w + ci * CHUNK
        pltpu.sync_copy(idxj_hbm.at[pl.ds(base, CHUNK)], idxj_v)
        pltpu.sync_copy(idxi_hbm.at[pl.ds(base, CHUNK)], idxi_v)
        pltpu.sync_copy(rbf_hbm.at[pl.ds(base, CHUNK)], rbf_v)
        cj = pltpu.async_copy(tj_hbm.at[idxj_v], rows_j, sem)
        ci_ = pltpu.async_copy(ti_hbm.at[idxi_v], rows_i, sem)
        cj.wait()
        ci_.wait()

        def eb_body(eb, _):
            e0 = eb * EB
            accs = []
            for ep in range(EB):
                e = e0 + ep
                accs.append([
                    rows_j[e, pl.ds(cb * LANES, LANES)]
                    + rows_i[e, pl.ds(cb * LANES, LANES)]
                    for cb in range(D // LANES)
                ])
            rbf_rows = [rbf_v[e0 + ep, :] for ep in range(EB)]
            for k in range(NR):
                wrk = [wr_v[k, pl.ds(cb * LANES, LANES)] for cb in range(D // LANES)]
                ksplat = jnp.full((LANES,), k, dtype=jnp.int32)
                for ep in range(EB):
                    s = _lane_broadcast(rbf_rows[ep], ksplat)
                    for cb in range(D // LANES):
                        accs[ep][cb] = accs[ep][cb] + s * wrk[cb]
            for ep in range(EB):
                e = e0 + ep
                for cb in range(D // LANES):
                    t = accs[ep][cb]
                    out_v[e, pl.ds(cb * LANES, LANES)] = t / (1.0 + jnp.exp(-t))
            return 0

        lax.fori_loop(0, CHUNK // EB, eb_body, 0)
        pltpu.sync_copy(out_v, out_hbm.at[pl.ds(base, CHUNK)])
        return 0

    lax.fori_loop(0, N_CHUNKS, chunk_body, 0)


@functools.partial(jax.jit, static_argnames=())
def _edge_kernel(tj, ti, wr, rbf, idx_j, idx_i):
    mesh = plsc.VectorSubcoreMesh(core_axis_name="c", subcore_axis_name="s")
    return pl.kernel(
        _edge_body,
        out_type=jax.ShapeDtypeStruct((N_EDGES, D), jnp.float32),
        mesh=mesh,
        scratch_types=[
            pltpu.VMEM((NR, D), jnp.float32),
            pltpu.VMEM((CHUNK,), jnp.int32),
            pltpu.VMEM((CHUNK,), jnp.int32),
            pltpu.VMEM((CHUNK, NR), jnp.float32),
            pltpu.VMEM((CHUNK, D), jnp.float32),
            pltpu.VMEM((CHUNK, D), jnp.float32),
            pltpu.VMEM((CHUNK, D), jnp.float32),
            pltpu.SemaphoreType.DMA,
        ],
    )(tj, ti, wr, rbf, idx_j, idx_i)


def kernel(x, rbf, idx_i, idx_j, W_rbf, W_edge, b_edge):
    tj, ti, wr = _precompute(x, W_rbf, W_edge, b_edge)
    idx_i = idx_i.astype(jnp.int32)
    idx_j = idx_j.astype(jnp.int32)
    return _edge_kernel(tj, ti, wr, rbf, idx_j, idx_i)


# speedup vs baseline: 3.5294x; 3.4733x over previous
import functools

import jax
import jax.numpy as jnp
from jax import lax
from jax.experimental import pallas as pl
from jax.experimental.pallas import tpu as pltpu
from jax.experimental.pallas import tpu_sc as plsc

N_NODES = 10000
N_EDGES = 320000
D = 128
NR = 16

NC = 2
NS = 16
NW = NC * NS
EPW = N_EDGES // NW
CHUNK = 40
N_PAIRS = EPW // (2 * CHUNK)
EB = 4
LANES = 16
NCB = D // LANES

ACC_BLOCK = 8000


def _precompute_body(x_ref, wrbf_ref, wedge_ref, b_ref, idxj_ref, idxi_ref,
                     t_ref, wr_ref, idx_ref):
    x = x_ref[...]
    t_ref[0:N_NODES, :] = jnp.dot(x, wedge_ref[0:D, :],
                                  preferred_element_type=jnp.float32)
    t_ref[N_NODES:2 * N_NODES, :] = (
        jnp.dot(x, wedge_ref[D:2 * D, :], preferred_element_type=jnp.float32)
        + b_ref[...]
    )
    wr_ref[...] = jnp.dot(wrbf_ref[...], wedge_ref[2 * D:3 * D, :],
                          preferred_element_type=jnp.float32)
    idx_ref[0] = idxj_ref[...]
    idx_ref[1] = idxi_ref[...] + N_NODES


def _precompute(x, W_rbf, W_edge, b_edge, idx_j, idx_i):
    return pl.pallas_call(
        _precompute_body,
        out_shape=[
            jax.ShapeDtypeStruct((2 * N_NODES, D), jnp.float32),
            jax.ShapeDtypeStruct((NR, D), jnp.float32),
            jax.ShapeDtypeStruct((2, N_EDGES // D, D), jnp.int32),
        ],
    )(x, W_rbf, W_edge, b_edge.reshape(1, D),
      idx_j.reshape(N_EDGES // D, D), idx_i.reshape(N_EDGES // D, D))


def _acc_body(rbf_ref, wr_ref, acc_ref):
    acc_ref[...] = jnp.dot(rbf_ref[...], wr_ref[...],
                           preferred_element_type=jnp.float32)


def _acc_matmul(rbf, wr):
    return pl.pallas_call(
        _acc_body,
        grid=(N_EDGES // ACC_BLOCK,),
        in_specs=[
            pl.BlockSpec((ACC_BLOCK, NR), lambda i: (i, 0)),
            pl.BlockSpec((NR, D), lambda i: (0, 0)),
        ],
        out_specs=pl.BlockSpec((ACC_BLOCK, D), lambda i: (i, 0)),
        out_shape=jax.ShapeDtypeStruct((N_EDGES, D), jnp.float32),
    )(rbf, wr)


def _edge_body(t_hbm, acc_hbm, idx_hbm, out_hbm,
               idx_v0, idx_v1, rows_j, rows_i, acc_v, out_v,
               sem_g0, sem_g1, sem_a0, sem_a1, sem_o0, sem_o1):
    wid = lax.axis_index("s") * NC + lax.axis_index("c")
    base_w = wid * EPW
    sem_g = (sem_g0, sem_g1)
    sem_a = (sem_a0, sem_a1)
    sem_o = (sem_o0, sem_o1)
    rows = ((rows_j.at[0], rows_i.at[0]), (rows_j.at[1], rows_i.at[1]))
    accb = (acc_v.at[0], acc_v.at[1])
    outb = (out_v.at[0], out_v.at[1])

    pltpu.sync_copy(idx_hbm.at[0, wid, 0, :], idx_v0)
    pltpu.sync_copy(idx_hbm.at[1, wid, 0, :], idx_v1)

    def issue_in(c, b):
        off = c * CHUNK
        pltpu.async_copy(t_hbm.at[idx_v0.at[pl.ds(off, CHUNK)]],
                         rows[b][0], sem_g[b])
        pltpu.async_copy(t_hbm.at[idx_v1.at[pl.ds(off, CHUNK)]],
                         rows[b][1], sem_g[b])
        pltpu.async_copy(acc_hbm.at[pl.ds(base_w + off, CHUNK)],
                         accb[b], sem_a[b])

    def wait_in(b):
        pltpu.make_async_copy(t_hbm.at[idx_v0.at[pl.ds(0, CHUNK)]],
                              rows[b][0], sem_g[b]).wait()
        pltpu.make_async_copy(t_hbm.at[idx_v1.at[pl.ds(0, CHUNK)]],
                              rows[b][1], sem_g[b]).wait()
        pltpu.make_async_copy(acc_hbm.at[pl.ds(0, CHUNK)],
                              accb[b], sem_a[b]).wait()

    def wait_out(b):
        pltpu.make_async_copy(outb[b], out_hbm.at[pl.ds(0, CHUNK)],
                              sem_o[b]).wait()

    def compute_store(c, b):
        rj, ri = rows[b]
        av, ov = accb[b], outb[b]

        def eb_body(i, _):
            e0 = i * EB
            for ep in range(EB):
                e = e0 + ep
                for cb in range(NCB):
                    sl = pl.ds(cb * LANES, LANES)
                    t = rj[e, sl] + ri[e, sl] + av[e, sl]
                    ov[e, sl] = t / (1.0 + jnp.exp(-t))
            return 0

        lax.fori_loop(0, CHUNK // EB, eb_body, 0)
        pltpu.async_copy(ov, out_hbm.at[pl.ds(base_w + c * CHUNK, CHUNK)],
                         sem_o[b])

    issue_in(0, 0)
    issue_in(1, 1)

    def pair_body(p, _):
        c0 = 2 * p
        for b in (0, 1):
            c = c0 + b
            wait_in(b)

            @pl.when(p > 0)
            def _():
                wait_out(b)

            compute_store(c, b)

            @pl.when(p < N_PAIRS - 1)
            def _():
                issue_in(c + 2, b)

        return 0

    lax.fori_loop(0, N_PAIRS, pair_body, 0)
    wait_out(0)
    wait_out(1)


def _edge_kernel(t, acc, idx_cat):
    mesh = plsc.VectorSubcoreMesh(core_axis_name="c", subcore_axis_name="s")
    return pl.kernel(
        _edge_body,
        out_type=jax.ShapeDtypeStruct((N_EDGES, D), jnp.float32),
        mesh=mesh,
        scratch_types=[
            pltpu.VMEM((EPW,), jnp.int32),
            pltpu.VMEM((EPW,), jnp.int32),
            pltpu.VMEM((2, CHUNK, D), jnp.float32),
            pltpu.VMEM((2, CHUNK, D), jnp.float32),
            pltpu.VMEM((2, CHUNK, D), jnp.float32),
            pltpu.VMEM((2, CHUNK, D), jnp.float32),
        ] + [pltpu.SemaphoreType.DMA] * 6,
    )(t, acc, idx_cat)


def kernel(x, rbf, idx_i, idx_j, W_rbf, W_edge, b_edge):
    idx_i = idx_i.astype(jnp.int32)
    idx_j = idx_j.astype(jnp.int32)
    t, wr, idx_cat = _precompute(x, W_rbf, W_edge, b_edge, idx_j, idx_i)
    acc = _acc_matmul(rbf, wr)
    return _edge_kernel(t, acc, idx_cat.reshape(2, NW, 1, EPW))
